# baseline (speedup 1.0000x reference)
import jax
import jax.numpy as jnp
import numpy as np
from jax import lax
from jax.experimental import pallas as pl
from jax.experimental.pallas import tpu as pltpu

_LEGAL_ISO = np.array(
    [50, 64, 80, 100, 125, 160, 200, 250, 320, 400, 500, 640, 800, 1000,
     1250, 1600, 2000, 2500, 3200, 4000, 5000, 6400, 8000, 10000, 12800,
     16000, 20000, 25600, 32000, 40000, 51200], dtype=np.float32)
_N_TBL = 31


def _compute_scale(params_row):
    tbl = params_row[0:1, 0:_N_TBL]
    cam_row = params_row[0:1, _N_TBL:2 * _N_TBL]
    gain = params_row[0, 2 * _N_TBL]
    iso_v = params_row[0, 2 * _N_TBL + 1]
    lt = (tbl < iso_v).astype(jnp.int32)
    le = (tbl <= iso_v).astype(jnp.int32)
    l_idx = jnp.minimum(jnp.sum(lt), _N_TBL - 1)
    r_idx = jnp.minimum(jnp.sum(le), _N_TBL - 1)
    iota = lax.broadcasted_iota(jnp.int32, (1, _N_TBL), 1)
    sel_l = (iota == l_idx)
    sel_r = (iota == r_idx)
    zf = jnp.zeros((1, _N_TBL), jnp.float32)
    iso_l = jnp.sum(jnp.where(sel_l, tbl, zf))
    iso_r = jnp.sum(jnp.where(sel_r, tbl, zf))
    cam_l = jnp.exp(jnp.sum(jnp.where(sel_l, cam_row, zf)))
    cam_r = jnp.exp(jnp.sum(jnp.where(sel_r, cam_row, zf)))
    denom = iso_r - iso_l
    safe_denom = jnp.where(denom != 0, denom, jnp.float32(1.0))
    cam = jnp.where(denom != 0,
                    ((iso_v - iso_l) * cam_r + (iso_r - iso_v) * cam_l) / safe_denom,
                    cam_l)
    return jnp.exp(cam * gain) * iso_v


def _gain_iso_kernel(params_ref, x_ref, z_ref, logdet_ref, smem_ref, *, n_chw, n_batch):
    @pl.when(pl.program_id(1) == 0)
    def _init():
        scale = _compute_scale(params_ref[...])
        smem_ref[0] = jnp.float32(1.0) / scale
        val = -jnp.float32(n_chw) * jnp.log(scale)
        logdet_ref[...] = jnp.zeros((1, n_batch), jnp.float32) + val

    z_ref[...] = x_ref[...] * smem_ref[0]


def kernel(x, cam_param, gain_params, iso):
    B, C, H, W = x.shape
    n_chw = C * H * W
    total = B * n_chw
    ncols = 4096
    nrows = total // ncols
    block_rows = 256
    grid = nrows // block_rows

    iso_f = jnp.asarray(iso, jnp.float32)
    gain_f = jnp.asarray(gain_params, jnp.float32)
    params_row = jnp.concatenate(
        [jnp.asarray(_LEGAL_ISO), cam_param.astype(jnp.float32),
         gain_f[None], iso_f[None]]).reshape(1, 2 * _N_TBL + 2)

    x2d = x.reshape(nrows, ncols)

    import functools
    body = functools.partial(_gain_iso_kernel, n_chw=n_chw, n_batch=B)
    outer = 2
    inner = grid // outer
    z2d, logdet = pl.pallas_call(
        body,
        grid=(outer, inner),
        in_specs=[
            pl.BlockSpec((1, 2 * _N_TBL + 2), lambda o, i: (0, 0)),
            pl.BlockSpec((block_rows, ncols), lambda o, i: (o * inner + i, 0)),
        ],
        out_specs=[
            pl.BlockSpec((block_rows, ncols), lambda o, i: (o * inner + i, 0)),
            pl.BlockSpec((1, B), lambda o, i: (0, 0)),
        ],
        out_shape=[
            jax.ShapeDtypeStruct((nrows, ncols), jnp.float32),
            jax.ShapeDtypeStruct((1, B), jnp.float32),
        ],
        scratch_shapes=[pltpu.SMEM((1,), jnp.float32)],
        compiler_params=pltpu.CompilerParams(
            dimension_semantics=("parallel", "arbitrary")),
    )(params_row, x2d)

    return z2d.reshape(B, C, H, W), logdet.reshape(B)


# speedup vs baseline: 3.3847x; 3.3847x over previous
import functools

import jax
import jax.numpy as jnp
import numpy as np
from jax import lax
from jax.experimental import pallas as pl
from jax.experimental.pallas import tpu as pltpu
from jax.experimental.pallas import tpu_sc as plsc

_LEGAL_ISO = np.array(
    [50, 64, 80, 100, 125, 160, 200, 250, 320, 400, 500, 640, 800, 1000,
     1250, 1600, 2000, 2500, 3200, 4000, 5000, 6400, 8000, 10000, 12800,
     16000, 20000, 25600, 32000, 40000, 51200], dtype=np.float32)
_N_TBL = 31
_L = 16


def _compute_scale(params_row):
    tbl = params_row[0:1, 0:_N_TBL]
    cam_row = params_row[0:1, _N_TBL:2 * _N_TBL]
    gain = params_row[0, 2 * _N_TBL]
    iso_v = params_row[0, 2 * _N_TBL + 1]
    lt = (tbl < iso_v).astype(jnp.int32)
    le = (tbl <= iso_v).astype(jnp.int32)
    l_idx = jnp.minimum(jnp.sum(lt), _N_TBL - 1)
    r_idx = jnp.minimum(jnp.sum(le), _N_TBL - 1)
    iota = lax.broadcasted_iota(jnp.int32, (1, _N_TBL), 1)
    sel_l = (iota == l_idx)
    sel_r = (iota == r_idx)
    zf = jnp.zeros((1, _N_TBL), jnp.float32)
    iso_l = jnp.sum(jnp.where(sel_l, tbl, zf))
    iso_r = jnp.sum(jnp.where(sel_r, tbl, zf))
    cam_l = jnp.exp(jnp.sum(jnp.where(sel_l, cam_row, zf)))
    cam_r = jnp.exp(jnp.sum(jnp.where(sel_r, cam_row, zf)))
    denom = iso_r - iso_l
    safe_denom = jnp.where(denom != 0, denom, jnp.float32(1.0))
    cam = jnp.where(denom != 0,
                    ((iso_v - iso_l) * cam_r + (iso_r - iso_v) * cam_l) / safe_denom,
                    cam_l)
    return jnp.exp(cam * gain) * iso_v


def _rescale_kernel(params_ref, x_ref, z_ref, smem_ref):
    @pl.when(pl.program_id(1) == 0)
    def _init():
        smem_ref[0] = jnp.float32(1.0) / _compute_scale(params_ref[...])

    z_ref[...] = x_ref[...] * smem_ref[0]


def _log_f32(v):
    bits = plsc.bitcast(v, jnp.int32)
    e = ((bits >> 23) & 0xFF) - 127
    m = plsc.bitcast((bits & 0x7FFFFF) | 0x3F800000, jnp.float32)
    big = m > jnp.float32(1.4142135623730951)
    m = jnp.where(big, m * jnp.float32(0.5), m)
    e = jnp.where(big, e + 1, e)
    t = (m - 1.0) / (m + 1.0)
    t2 = t * t
    p = jnp.float32(1.0 / 9.0)
    p = p * t2 + jnp.float32(1.0 / 7.0)
    p = p * t2 + jnp.float32(1.0 / 5.0)
    p = p * t2 + jnp.float32(1.0 / 3.0)
    p = p * t2 + jnp.float32(1.0)
    ln2 = jnp.float32(0.6931471805599453)
    return e.astype(jnp.float32) * ln2 + jnp.float32(2.0) * t * p


def _logdet_sc_kernel(params_hbm, out_hbm, pvmem, ovmem, *, n_chw):
    cid = lax.axis_index("c")
    sid = lax.axis_index("s")

    @pl.when(jnp.logical_and(cid == 0, sid == 0))
    def _only_tile0():
        pltpu.sync_copy(params_hbm, pvmem)
        t0 = pvmem[pl.ds(0, _L)]
        t1 = pvmem[pl.ds(_L, _L)]
        gain = pvmem[pl.ds(4 * _L, _L)]
        iso_v = pvmem[pl.ds(5 * _L, _L)]
        l_cnt = (plsc.all_reduce_population_count(t0 < iso_v)
                 + plsc.all_reduce_population_count(t1 < iso_v))
        r_cnt = (plsc.all_reduce_population_count(t0 <= iso_v)
                 + plsc.all_reduce_population_count(t1 <= iso_v))
        l_idx = jnp.minimum(l_cnt, _N_TBL - 1)
        r_idx = jnp.minimum(r_cnt, _N_TBL - 1)
        iso_l = plsc.load_gather(pvmem, [l_idx])
        iso_r = plsc.load_gather(pvmem, [r_idx])
        cam_l = jnp.exp(plsc.load_gather(pvmem, [l_idx + 2 * _L]))
        cam_r = jnp.exp(plsc.load_gather(pvmem, [r_idx + 2 * _L]))
        denom = iso_r - iso_l
        nz = denom != jnp.float32(0.0)
        safe_denom = jnp.where(nz, denom, jnp.float32(1.0))
        cam = jnp.where(nz,
                        ((iso_v - iso_l) * cam_r + (iso_r - iso_v) * cam_l) / safe_denom,
                        cam_l)
        log_scale = cam * gain + _log_f32(iso_v)
        ovmem[...] = -jnp.float32(n_chw) * log_scale
        pltpu.sync_copy(ovmem, out_hbm)


def kernel(x, cam_param, gain_params, iso):
    B, C, H, W = x.shape
    n_chw = C * H * W
    total = B * n_chw
    ncols = W
    nrows = total // ncols
    block_rows = 4096
    grid = nrows // block_rows

    iso_f = jnp.asarray(iso, jnp.float32)
    gain_f = jnp.asarray(gain_params, jnp.float32)
    cam_f = cam_param.astype(jnp.float32)
    params_row = jnp.concatenate(
        [jnp.asarray(_LEGAL_ISO), cam_f, gain_f[None], iso_f[None]]
    ).reshape(1, 2 * _N_TBL + 2)

    inf_pad = jnp.full((1,), np.inf, jnp.float32)
    params_sc = jnp.concatenate([
        jnp.asarray(_LEGAL_ISO), inf_pad,
        cam_f, jnp.zeros((1,), jnp.float32),
        jnp.full((_L,), gain_f, jnp.float32),
        jnp.full((_L,), iso_f, jnp.float32),
    ])

    x2d = x.reshape(nrows, ncols)
    outer = 4
    inner = grid // outer
    z2d = pl.pallas_call(
        _rescale_kernel,
        grid=(outer, inner),
        in_specs=[
            pl.BlockSpec((1, 2 * _N_TBL + 2), lambda o, i: (0, 0)),
            pl.BlockSpec((block_rows, ncols), lambda o, i: (o * inner + i, 0)),
        ],
        out_specs=pl.BlockSpec((block_rows, ncols), lambda o, i: (o * inner + i, 0)),
        out_shape=jax.ShapeDtypeStruct((nrows, ncols), jnp.float32),
        scratch_shapes=[pltpu.SMEM((1,), jnp.float32)],
        compiler_params=pltpu.CompilerParams(
            dimension_semantics=("parallel", "arbitrary")),
    )(params_row, x2d)

    mesh = plsc.VectorSubcoreMesh(core_axis_name="c", subcore_axis_name="s",
                                  num_cores=1, num_subcores=1)
    sc_body = functools.partial(_logdet_sc_kernel, n_chw=n_chw)
    logdet = pl.kernel(
        sc_body,
        mesh=mesh,
        out_type=jax.ShapeDtypeStruct((B,), jnp.float32),
        scratch_types=[
            pltpu.VMEM((6 * _L,), jnp.float32),
            pltpu.VMEM((B,), jnp.float32),
        ],
        compiler_params=pltpu.CompilerParams(needs_layout_passes=False),
    )(params_sc)

    return z2d.reshape(B, C, H, W), logdet
